# baseline (speedup 1.0000x reference)
import functools

import jax
import jax.numpy as jnp
from jax import lax
from jax.experimental import pallas as pl
from jax.experimental.pallas import tpu as pltpu
from jax.experimental.pallas import tpu_sc as plsc

_N = 100000
_D = 768
_NC = 2
_NS = 16
_NW = _NC * _NS
_C = 3136
_NP = _C * _NW
_GRP = _C // 16
_K = 32
_NB = _C // _K
_NRING = 4
_CPAD = (_NB + _NRING + 3) // 4 * 4 * _K + _NRING * _K

_mesh = plsc.VectorSubcoreMesh(core_axis_name="c", subcore_axis_name="s")

_GDN = lax.GatherDimensionNumbers(
    offset_dims=(), collapsed_slice_dims=(0,), start_index_map=(0,))


def _take16(x, idx):
    return lax.gather(x, idx[:, None], _GDN, slice_sizes=(1,),
                      mode=lax.GatherScatterMode.PROMISE_IN_BOUNDS)


def _splat_sum16(x):
    iota = lax.iota(jnp.int32, 16)
    for k in (1, 2, 4, 8):
        x = x + _take16(x, iota ^ k)
    return x


@functools.partial(
    pl.kernel,
    out_type=[
        jax.ShapeDtypeStruct((_NW, 16), jnp.float32),
        jax.ShapeDtypeStruct((_NC, 16), jnp.float32),
    ],
    mesh=_mesh,
    scratch_types=[
        pltpu.VMEM((_C,), jnp.float32),
        pltpu.VMEM((_C,), jnp.float32),
        pltpu.VMEM((_CPAD,), jnp.int32),
        pltpu.VMEM((_K, _D), jnp.float32),
        pltpu.VMEM((_K, _D), jnp.float32),
        pltpu.VMEM((_K, _D), jnp.float32),
        pltpu.VMEM((_K, _D), jnp.float32),
        pltpu.VMEM((_D,), jnp.float32),
        pltpu.VMEM((_D,), jnp.float32),
        pltpu.VMEM((_D,), jnp.float32),
        pltpu.VMEM((16,), jnp.float32),
        pltpu.VMEM((16,), jnp.float32),
        pltpu.VMEM_SHARED((_NS, _D), jnp.float32),
        pltpu.SemaphoreType.DMA,
        pltpu.SemaphoreType.DMA,
        pltpu.SemaphoreType.DMA,
        pltpu.SemaphoreType.DMA,
    ],
)
def _sc_kernel(pv_hbm, s_hbm, w_hbm, bo_hbm, emb_hbm,
               ps_out, u_out,
               pv_v, s_v, idx_v, buf0_v, buf1_v, buf2_v, buf3_v,
               u_v, e0_v, w_v, st16_v, bo_v,
               shared, sem0, sem1, sem2, sem3):
    c = lax.axis_index("c")
    s = lax.axis_index("s")
    wid = c * _NS + s
    base = pl.multiple_of(wid * _C, _C)

    pltpu.sync_copy(pv_hbm.at[pl.ds(base, _C)], pv_v)
    pltpu.sync_copy(s_hbm.at[pl.ds(base, _C)], s_v)
    pltpu.sync_copy(bo_hbm, bo_v)
    pltpu.sync_copy(emb_hbm.at[0], e0_v)

    def zfill(g, carry):
        idx_v[pl.ds(_C + g * 16, 16)] = jnp.zeros((16,), jnp.int32)
        return carry
    lax.fori_loop(0, (_CPAD - _C) // 16, zfill, 0)

    def grp(g, carry):
        ps_vec, cnt_vec = carry
        v = pv_v[pl.ds(g * 16, 16)]
        sc = s_v[pl.ds(g * 16, 16)]
        gidx = base + g * 16 + lax.iota(jnp.int32, 16)
        idx_v[pl.ds(g * 16, 16)] = jnp.where(v == 1.0, gidx, 0)
        return (ps_vec + v * sc, cnt_vec + v)

    ps_vec, cnt_vec = lax.fori_loop(
        0, _GRP, grp,
        (jnp.zeros((16,), jnp.float32), jnp.zeros((16,), jnp.float32)))

    st16_v[...] = ps_vec + cnt_vec * bo_v[...]
    pltpu.sync_copy(st16_v, ps_out.at[wid])

    bufs = (buf0_v, buf1_v, buf2_v, buf3_v)
    sems = (sem0, sem1, sem2, sem3)

    def fire(batch_i, ring_i):
        pltpu.async_copy(
            emb_hbm.at[idx_v.at[pl.ds(pl.multiple_of(batch_i * _K, _K), _K)]],
            bufs[ring_i], sems[ring_i])

    def drain(ring_i):
        pltpu.make_async_copy(
            emb_hbm.at[pl.ds(0, _K)], bufs[ring_i], sems[ring_i]).wait()

    for i in range(_NRING):
        fire(i, i)

    def quad(t, acc):
        for i in range(_NRING):
            drain(i)

            def row_add(r, a):
                return tuple(
                    x + bufs[i][r, pl.ds(j * 16, 16)]
                    for j, x in enumerate(a))

            acc = lax.fori_loop(0, _K, row_add, acc)
            fire(t * _NRING + _NRING + i, i)
        return acc

    nquads = _NB // _NRING
    acc = lax.fori_loop(0, nquads, quad,
                        tuple(jnp.zeros((16,), jnp.float32)
                              for _ in range(_D // 16)))
    for i in range(_NRING):
        drain(i)

        def row_add_t(r, a):
            return tuple(
                x + bufs[i][r, pl.ds(j * 16, 16)] for j, x in enumerate(a))

        if i < _NB - nquads * _NRING:
            acc = lax.fori_loop(0, _K, row_add_t, acc)

    ndum = _splat_sum16(cnt_vec)
    dums = jnp.float32(_C) - ndum
    for j in range(_D // 16):
        u_v[pl.ds(j * 16, 16)] = acc[j] - dums * e0_v[pl.ds(j * 16, 16)]

    pltpu.sync_copy(u_v, shared.at[s])
    plsc.subcore_barrier()

    @pl.when(s == 0)
    def _():
        pltpu.sync_copy(w_hbm, w_v)

        def core_fold(k, acc16):
            pltpu.sync_copy(shared.at[k], u_v)

            def dot(j, a):
                return a + u_v[pl.ds(j * 16, 16)] * w_v[pl.ds(j * 16, 16)]

            return lax.fori_loop(0, _D // 16, dot, acc16)

        st16_v[...] = lax.fori_loop(0, _NS, core_fold,
                                    jnp.zeros((16,), jnp.float32))
        pltpu.sync_copy(st16_v, u_out.at[c])


def kernel(phrase_vector, score, W, b, offset, emb_table):
    pv = jnp.pad(phrase_vector[0], (0, _NP - _N))
    sc = jnp.pad(score[0], (0, _NP - _N))
    bo = jnp.full((16,), b[0] + offset[0], jnp.float32)
    ps_out, u_out = _sc_kernel(pv, sc, W.reshape(_D), bo, emb_table)
    return (jnp.sum(ps_out) + jnp.sum(u_out)).reshape(1, 1)


# speedup vs baseline: 20.8519x; 20.8519x over previous
import jax
import jax.numpy as jnp
from jax.experimental import pallas as pl
from jax.experimental.pallas import tpu as pltpu

_N = 100000
_D = 768
_BN = 5000
_G = _N // _BN


def _body(pv_ref, s_ref, w_ref, b_ref, off_ref, emb_ref, out_ref, acc_ref, ps_ref):
    i = pl.program_id(0)

    @pl.when(i == 0)
    def _():
        acc_ref[...] = jnp.zeros_like(acc_ref)
        pv_all = pv_ref[...]
        ps_ref[0] = jnp.sum(pv_all * s_ref[...])
        ps_ref[1] = jnp.sum(pv_all)

    pv = pv_ref[pl.ds(i, 1), :]
    acc_ref[...] += jnp.dot(pv, emb_ref[...], preferred_element_type=jnp.float32)

    @pl.when(i == _G - 1)
    def _():
        total = jnp.dot(acc_ref[...], w_ref[...], preferred_element_type=jnp.float32)
        out_ref[...] = total + ps_ref[0] + ps_ref[1] * (b_ref[0] + off_ref[0])


def kernel(phrase_vector, score, W, b, offset, emb_table):
    return pl.pallas_call(
        _body,
        grid=(_G,),
        in_specs=[
            pl.BlockSpec((_G, _BN), lambda i: (0, 0)),
            pl.BlockSpec((_G, _BN), lambda i: (0, 0)),
            pl.BlockSpec((_D, 1), lambda i: (0, 0)),
            pl.BlockSpec(memory_space=pltpu.SMEM),
            pl.BlockSpec(memory_space=pltpu.SMEM),
            pl.BlockSpec((_BN, _D), lambda i: (i, 0)),
        ],
        out_specs=pl.BlockSpec((1, 1), lambda i: (0, 0)),
        out_shape=jax.ShapeDtypeStruct((1, 1), jnp.float32),
        scratch_shapes=[
            pltpu.VMEM((1, _D), jnp.float32),
            pltpu.SMEM((2,), jnp.float32),
        ],
    )(phrase_vector.reshape(_G, _BN), score.reshape(_G, _BN), W, b, offset, emb_table)
